# baseline (speedup 1.0000x reference)
import functools

import jax
import jax.numpy as jnp
import numpy as np
from jax import lax
from jax.experimental import pallas as pl
from jax.experimental.pallas import tpu as pltpu
from jax.experimental.pallas import tpu_sc as plsc

BATCH = 4096
SEQ = 200
DIM = 32
NITEMS = 1000000
NW = 32
NCOLS = NITEMS // 128
COL_TAIL = NITEMS - NCOLS * 128
A_ITERS = 122
A_EXTRA = NCOLS - 2 * A_ITERS * NW
B_PER_W = BATCH // NW

_J = np.arange(16)

_LUT_A = np.zeros((8 + DIM, 128), np.int32)
for _k in range(8):
    _LUT_A[_k, :16] = 4 * _k + _J // 4
for _d in range(DIM):
    _LUT_A[8 + _d, :16] = (_J % 4) * 32 + _d

_LUT_B = np.zeros((3 + B_PER_W, 16), np.int32)
_LUT_B[0] = _J // 8
_LUT_B[1] = 2 + _J // 8
_LUT_B[2] = _J % 8
for _bi in range(B_PER_W):
    _LUT_B[3 + _bi] = _bi


def _detile_table():
    mesh = plsc.VectorSubcoreMesh(core_axis_name="c", subcore_axis_name="s")

    @functools.partial(
        pl.kernel,
        mesh=mesh,
        out_type=jax.ShapeDtypeStruct((NITEMS // 4, 128), jnp.float32),
        compiler_params=pltpu.CompilerParams(use_tc_tiling_on_sc=True, needs_layout_passes=False),
        scratch_types=[
            pltpu.VMEM((8 + DIM, 128), jnp.int32),
            pltpu.VMEM((DIM, 128), jnp.float32),
            pltpu.VMEM((DIM, 128), jnp.float32),
            pltpu.VMEM((DIM, 128), jnp.float32),
            pltpu.VMEM((DIM, 128), jnp.float32),
            pltpu.SemaphoreType.DMA,
            pltpu.SemaphoreType.DMA,
            pltpu.SemaphoreType.DMA,
            pltpu.SemaphoreType.DMA,
        ],
    )
    def ka(tt_hbm, lut_hbm, tail_hbm, out_hbm, lut_v, tb0, tb1, ob0, ob1,
           gs0, gs1, ss0, ss1):
        wid = lax.axis_index("s") * 2 + lax.axis_index("c")
        tbufs = (tb0, tb1)
        obufs = (ob0, ob1)
        gsems = (gs0, gs1)
        ssems = (ss0, ss1)

        pltpu.sync_copy(lut_hbm, lut_v)
        rowv = [lut_v[k, pl.ds(0, 16)] for k in range(8)]

        def col_of(t, b):
            return (2 * t + b) * NW + wid

        def fetch(t, b):
            col = col_of(t, b)
            return pltpu.make_async_copy(
                tt_hbm.at[:, pl.ds(col * 128, 128)], tbufs[b], gsems[b]
            )

        def flush(t, b):
            col = col_of(t, b)
            return pltpu.make_async_copy(
                obufs[b], out_hbm.at[pl.ds(col * 32, DIM)], ssems[b]
            )

        def transpose_col(tbuf, obuf, nk):
            def dbody(d, carry):
                cvec = lut_v[8 + d, pl.ds(0, 16)]
                for k in range(nk):
                    v = tbuf[d, pl.ds(16 * k, 16)]
                    plsc.store_scatter(obuf, [rowv[k], cvec], v)
                return carry

            lax.fori_loop(0, DIM, dbody, 0)

        fetch(0, 0).start()
        fetch(0, 1).start()

        for b in range(2):
            fetch(0, b).wait()
            transpose_col(tbufs[b], obufs[b], 8)
            fetch(1, b).start()
            flush(0, b).start()

        def tbody(t, carry):
            for b in range(2):
                fetch(t, b).wait()
                flush(t - 1, b).wait()
                transpose_col(tbufs[b], obufs[b], 8)
                fetch(t + 1, b).start()
                flush(t, b).start()
            return carry

        lax.fori_loop(1, A_ITERS - 1, tbody, 0)

        for b in range(2):
            t = A_ITERS - 1
            fetch(t, b).wait()
            flush(t - 1, b).wait()
            transpose_col(tbufs[b], obufs[b], 8)
            flush(t, b).start()
        for b in range(2):
            flush(A_ITERS - 1, b).wait()

        @pl.when(wid < A_EXTRA)
        def _():
            col = 2 * A_ITERS * NW + wid
            pltpu.sync_copy(tt_hbm.at[:, pl.ds(col * 128, 128)], tb0)
            transpose_col(tb0, ob0, 8)
            pltpu.sync_copy(ob0, out_hbm.at[pl.ds(col * 32, DIM)])

        @pl.when(wid == A_EXTRA)
        def _():
            pltpu.sync_copy(tail_hbm, ob1.at[pl.ds(0, COL_TAIL // 4)])
            pltpu.sync_copy(
                ob1.at[pl.ds(0, COL_TAIL // 4)],
                out_hbm.at[pl.ds(NCOLS * 32, COL_TAIL // 4)],
            )

    return ka


def _gather_add():
    mesh = plsc.VectorSubcoreMesh(core_axis_name="c", subcore_axis_name="s")

    @functools.partial(
        pl.kernel,
        mesh=mesh,
        out_type=jax.ShapeDtypeStruct((SEQ, 4, 32, 8, 128), jnp.float32),
        compiler_params=pltpu.CompilerParams(use_tc_tiling_on_sc=False, needs_layout_passes=False),
        scratch_types=[
            pltpu.VMEM((3 + B_PER_W, 16), jnp.int32),
            pltpu.VMEM((SEQ, B_PER_W), jnp.int32),
            pltpu.VMEM((SEQ, DIM), jnp.float32),
            pltpu.VMEM((B_PER_W, DIM), jnp.float32),
            pltpu.VMEM((B_PER_W, DIM), jnp.float32),
            pltpu.VMEM((4, 8, 128), jnp.float32),
            pltpu.VMEM((4, 8, 128), jnp.float32),
            pltpu.SemaphoreType.DMA,
            pltpu.SemaphoreType.DMA,
            pltpu.SemaphoreType.DMA,
            pltpu.SemaphoreType.DMA,
        ],
    )
    def kb(tab_hbm, xt_hbm, pos_hbm, lut_hbm, out_hbm, lut_v, idx_all,
           pos_v, gb0, gb1, ob0, ob1, gs0, gs1, ss0, ss1):
        wid = lax.axis_index("s") * 2 + lax.axis_index("c")
        gbufs = (gb0, gb1)
        obufs = (ob0, ob1)
        gsems = (gs0, gs1)
        ssems = (ss0, ss1)

        pltpu.sync_copy(lut_hbm, lut_v)
        pltpu.sync_copy(xt_hbm.at[:, pl.ds(wid * B_PER_W, B_PER_W)], idx_all)
        pltpu.sync_copy(pos_hbm, pos_v)

        dt0 = lut_v[0, pl.ds(0, 16)]
        dt1 = lut_v[1, pl.ds(0, 16)]
        dsv = lut_v[2, pl.ds(0, 16)]

        def fetch(l, b):
            return pltpu.make_async_copy(
                tab_hbm.at[idx_all.at[l]], gbufs[b], gsems[b]
            )

        def flush(l, b):
            return pltpu.make_async_copy(
                obufs[b], out_hbm.at[l, :, wid], ssems[b]
            )

        def process(l, gbuf, obuf):
            p0 = pos_v[l, pl.ds(0, 16)]
            p1 = pos_v[l, pl.ds(16, 16)]

            def bbody(bi, carry):
                bvec = lut_v[3 + bi, pl.ds(0, 16)]
                v0 = gbuf[bi, pl.ds(0, 16)] + p0
                v1 = gbuf[bi, pl.ds(16, 16)] + p1
                plsc.store_scatter(obuf, [dt0, dsv, bvec], v0)
                plsc.store_scatter(obuf, [dt1, dsv, bvec], v1)
                return carry

            lax.fori_loop(0, B_PER_W, bbody, 0)

        fetch(0, 0).start()
        fetch(1, 1).start()

        for b in range(2):
            fetch(b, b).wait()
            process(b, gbufs[b], obufs[b])
            fetch(b + 2, b).start()
            flush(b, b).start()

        def lbody(t, carry):
            for b in range(2):
                l = 2 * t + b
                fetch(l, b).wait()
                flush(l - 2, b).wait()
                process(l, gbufs[b], obufs[b])
                fetch(l + 2, b).start()
                flush(l, b).start()
            return carry

        lax.fori_loop(1, SEQ // 2 - 1, lbody, 0)

        for b in range(2):
            l = SEQ - 2 + b
            fetch(l, b).wait()
            flush(l - 2, b).wait()
            process(l, gbufs[b], obufs[b])
            flush(l, b).start()
        flush(SEQ - 2, 0).wait()
        flush(SEQ - 1, 1).wait()

    return kb


def kernel(x, item_emb_matrix, positional_emb):
    table_t = jnp.swapaxes(item_emb_matrix, 0, 1)
    lut_a = jnp.asarray(_LUT_A)
    tail_rm = item_emb_matrix[NCOLS * 128:].reshape(COL_TAIL // 4, 128)
    table_rm = _detile_table()(table_t, lut_a, tail_rm)
    table_flat = table_rm.reshape(NITEMS, DIM)
    x_t = jnp.swapaxes(x, 0, 1).astype(jnp.int32)
    lut_b = jnp.asarray(_LUT_B)
    out5 = _gather_add()(table_flat, x_t, positional_emb, lut_b)
    return out5.transpose(2, 4, 0, 1, 3).reshape(BATCH, SEQ, DIM)


# speedup vs baseline: 1.0089x; 1.0089x over previous
import functools

import jax
import jax.numpy as jnp
import numpy as np
from jax import lax
from jax.experimental import pallas as pl
from jax.experimental.pallas import tpu as pltpu
from jax.experimental.pallas import tpu_sc as plsc

BATCH = 4096
SEQ = 200
DIM = 32
NITEMS = 1000000
NW = 32
NCOLS = NITEMS // 128
COL_TAIL = NITEMS - NCOLS * 128
A_ITERS = 122
A_EXTRA = NCOLS - 2 * A_ITERS * NW
B_PER_W = BATCH // NW

_J = np.arange(16)

_LUT_A = np.zeros((8 + DIM, 128), np.int32)
for _k in range(8):
    _LUT_A[_k, :16] = 4 * _k + _J // 4
for _d in range(DIM):
    _LUT_A[8 + _d, :16] = (_J % 4) * 32 + _d

_LUT_B = np.zeros((3 + B_PER_W, 16), np.int32)
_LUT_B[0] = _J // 8
_LUT_B[1] = 2 + _J // 8
_LUT_B[2] = _J % 8
for _bi in range(B_PER_W):
    _LUT_B[3 + _bi] = _bi


def _detile_table():
    mesh = plsc.VectorSubcoreMesh(core_axis_name="c", subcore_axis_name="s")

    @functools.partial(
        pl.kernel,
        mesh=mesh,
        out_type=jax.ShapeDtypeStruct((NITEMS // 4, 128), jnp.float32),
        compiler_params=pltpu.CompilerParams(use_tc_tiling_on_sc=True, needs_layout_passes=False),
        scratch_types=[
            pltpu.VMEM((8 + DIM, 128), jnp.int32),
            pltpu.VMEM((DIM, 128), jnp.float32),
            pltpu.VMEM((DIM, 128), jnp.float32),
            pltpu.VMEM((DIM, 128), jnp.float32),
            pltpu.VMEM((DIM, 128), jnp.float32),
            pltpu.SemaphoreType.DMA,
            pltpu.SemaphoreType.DMA,
            pltpu.SemaphoreType.DMA,
            pltpu.SemaphoreType.DMA,
        ],
    )
    def ka(tt_hbm, lut_hbm, tail_hbm, out_hbm, lut_v, tb0, tb1, ob0, ob1,
           gs0, gs1, ss0, ss1):
        wid = lax.axis_index("s") * 2 + lax.axis_index("c")
        tbufs = (tb0, tb1)
        obufs = (ob0, ob1)
        gsems = (gs0, gs1)
        ssems = (ss0, ss1)

        pltpu.sync_copy(lut_hbm, lut_v)
        rowv = [lut_v[k, pl.ds(0, 16)] for k in range(8)]

        def col_of(t, b):
            return (2 * t + b) * NW + wid

        def fetch(t, b):
            col = col_of(t, b)
            return pltpu.make_async_copy(
                tt_hbm.at[:, pl.ds(col * 128, 128)], tbufs[b], gsems[b]
            )

        def flush(t, b):
            col = col_of(t, b)
            return pltpu.make_async_copy(
                obufs[b], out_hbm.at[pl.ds(col * 32, DIM)], ssems[b]
            )

        def transpose_col(tbuf, obuf, nk):
            def dbody(o, carry):
                for u in range(4):
                    d = o * 4 + u
                    cvec = lut_v[8 + d, pl.ds(0, 16)]
                    for k in range(nk):
                        v = tbuf[d, pl.ds(16 * k, 16)]
                        plsc.store_scatter(obuf, [rowv[k], cvec], v)
                return carry

            lax.fori_loop(0, DIM // 4, dbody, 0)

        fetch(0, 0).start()
        fetch(0, 1).start()

        for b in range(2):
            fetch(0, b).wait()
            transpose_col(tbufs[b], obufs[b], 8)
            fetch(1, b).start()
            flush(0, b).start()

        def tbody(t, carry):
            for b in range(2):
                fetch(t, b).wait()
                flush(t - 1, b).wait()
                transpose_col(tbufs[b], obufs[b], 8)
                fetch(t + 1, b).start()
                flush(t, b).start()
            return carry

        lax.fori_loop(1, A_ITERS - 1, tbody, 0)

        for b in range(2):
            t = A_ITERS - 1
            fetch(t, b).wait()
            flush(t - 1, b).wait()
            transpose_col(tbufs[b], obufs[b], 8)
            flush(t, b).start()
        for b in range(2):
            flush(A_ITERS - 1, b).wait()

        @pl.when(wid < A_EXTRA)
        def _():
            col = 2 * A_ITERS * NW + wid
            pltpu.sync_copy(tt_hbm.at[:, pl.ds(col * 128, 128)], tb0)
            transpose_col(tb0, ob0, 8)
            pltpu.sync_copy(ob0, out_hbm.at[pl.ds(col * 32, DIM)])

        @pl.when(wid == A_EXTRA)
        def _():
            pltpu.sync_copy(tail_hbm, ob1.at[pl.ds(0, COL_TAIL // 4)])
            pltpu.sync_copy(
                ob1.at[pl.ds(0, COL_TAIL // 4)],
                out_hbm.at[pl.ds(NCOLS * 32, COL_TAIL // 4)],
            )

    return ka


def _gather_add():
    mesh = plsc.VectorSubcoreMesh(core_axis_name="c", subcore_axis_name="s")

    @functools.partial(
        pl.kernel,
        mesh=mesh,
        out_type=jax.ShapeDtypeStruct((SEQ, 4, 32, 8, 128), jnp.float32),
        compiler_params=pltpu.CompilerParams(use_tc_tiling_on_sc=False, needs_layout_passes=False),
        scratch_types=[
            pltpu.VMEM((3 + B_PER_W, 16), jnp.int32),
            pltpu.VMEM((SEQ, B_PER_W), jnp.int32),
            pltpu.VMEM((SEQ, DIM), jnp.float32),
            pltpu.VMEM((B_PER_W, DIM), jnp.float32),
            pltpu.VMEM((B_PER_W, DIM), jnp.float32),
            pltpu.VMEM((B_PER_W, DIM), jnp.float32),
            pltpu.VMEM((B_PER_W, DIM), jnp.float32),
            pltpu.VMEM((4, 8, 128), jnp.float32),
            pltpu.VMEM((4, 8, 128), jnp.float32),
            pltpu.VMEM((4, 8, 128), jnp.float32),
            pltpu.VMEM((4, 8, 128), jnp.float32),
            pltpu.SemaphoreType.DMA,
            pltpu.SemaphoreType.DMA,
            pltpu.SemaphoreType.DMA,
            pltpu.SemaphoreType.DMA,
            pltpu.SemaphoreType.DMA,
            pltpu.SemaphoreType.DMA,
            pltpu.SemaphoreType.DMA,
            pltpu.SemaphoreType.DMA,
        ],
    )
    def kb(tab_hbm, xt_hbm, pos_hbm, lut_hbm, out_hbm, lut_v, idx_all,
           pos_v, gb0, gb1, gb2, gb3, ob0, ob1, ob2, ob3,
           gs0, gs1, gs2, gs3, ss0, ss1, ss2, ss3):
        wid = lax.axis_index("s") * 2 + lax.axis_index("c")
        gbufs = (gb0, gb1, gb2, gb3)
        obufs = (ob0, ob1, ob2, ob3)
        gsems = (gs0, gs1, gs2, gs3)
        ssems = (ss0, ss1, ss2, ss3)

        pltpu.sync_copy(lut_hbm, lut_v)
        pltpu.sync_copy(xt_hbm.at[:, pl.ds(wid * B_PER_W, B_PER_W)], idx_all)
        pltpu.sync_copy(pos_hbm, pos_v)

        dt0 = lut_v[0, pl.ds(0, 16)]
        dt1 = lut_v[1, pl.ds(0, 16)]
        dsv = lut_v[2, pl.ds(0, 16)]

        def fetch(l, b):
            return pltpu.make_async_copy(
                tab_hbm.at[idx_all.at[l]], gbufs[b], gsems[b]
            )

        def flush(l, b):
            return pltpu.make_async_copy(
                obufs[b], out_hbm.at[l, :, wid], ssems[b]
            )

        def process(l, gbuf, obuf):
            p0 = pos_v[l, pl.ds(0, 16)]
            p1 = pos_v[l, pl.ds(16, 16)]

            def bbody(b4, carry):
                for u in range(4):
                    bi = b4 * 4 + u
                    bvec = lut_v[3 + bi, pl.ds(0, 16)]
                    v0 = gbuf[bi, pl.ds(0, 16)] + p0
                    v1 = gbuf[bi, pl.ds(16, 16)] + p1
                    plsc.store_scatter(obuf, [dt0, dsv, bvec], v0)
                    plsc.store_scatter(obuf, [dt1, dsv, bvec], v1)
                return carry

            lax.fori_loop(0, B_PER_W // 4, bbody, 0)

        NB = 4
        for b in range(NB):
            fetch(b, b).start()

        for b in range(NB):
            fetch(b, b).wait()
            process(b, gbufs[b], obufs[b])
            fetch(b + NB, b).start()
            flush(b, b).start()

        def lbody(t, carry):
            for b in range(NB):
                l = NB * t + b
                fetch(l, b).wait()
                flush(l - NB, b).wait()
                process(l, gbufs[b], obufs[b])
                fetch(l + NB, b).start()
                flush(l, b).start()
            return carry

        lax.fori_loop(1, SEQ // NB - 1, lbody, 0)

        for b in range(NB):
            l = SEQ - NB + b
            fetch(l, b).wait()
            flush(l - NB, b).wait()
            process(l, gbufs[b], obufs[b])
            flush(l, b).start()
        for b in range(NB):
            flush(SEQ - NB + b, b).wait()

    return kb


def kernel(x, item_emb_matrix, positional_emb):
    table_t = jnp.swapaxes(item_emb_matrix, 0, 1)
    lut_a = jnp.asarray(_LUT_A)
    tail_rm = item_emb_matrix[NCOLS * 128:].reshape(COL_TAIL // 4, 128)
    table_rm = _detile_table()(table_t, lut_a, tail_rm)
    table_flat = table_rm.reshape(NITEMS, DIM)
    x_t = jnp.swapaxes(x, 0, 1).astype(jnp.int32)
    lut_b = jnp.asarray(_LUT_B)
    out5 = _gather_add()(table_flat, x_t, positional_emb, lut_b)
    return out5.transpose(2, 4, 0, 1, 3).reshape(BATCH, SEQ, DIM)


# speedup vs baseline: 1.2760x; 1.2647x over previous
import functools

import jax
import jax.numpy as jnp
from jax import lax
from jax.experimental import pallas as pl
from jax.experimental.pallas import tpu as pltpu
from jax.experimental.pallas import tpu_sc as plsc

BATCH = 4096
SEQ = 200
DIM = 32
ROWS = BATCH * SEQ
SEQ_PER_CHUNK = 4
CHUNK = SEQ * SEQ_PER_CHUNK
NBUF = 2


def _build(num_workers, rows_per_w, nchunk):
    mesh = plsc.VectorSubcoreMesh(core_axis_name="c", subcore_axis_name="s")

    @functools.partial(
        pl.kernel,
        mesh=mesh,
        out_type=jax.ShapeDtypeStruct((ROWS, DIM), jnp.float32),
        compiler_params=pltpu.CompilerParams(use_tc_tiling_on_sc=False),
        scratch_types=[
            pltpu.VMEM((rows_per_w,), jnp.int32),
            pltpu.VMEM((NBUF, CHUNK, DIM), jnp.float32),
            pltpu.VMEM((SEQ, DIM), jnp.float32),
            pltpu.SemaphoreType.DMA,
            pltpu.SemaphoreType.DMA,
            pltpu.SemaphoreType.DMA,
            pltpu.SemaphoreType.DMA,
        ],
    )
    def k(table_hbm, idx_hbm, pos_hbm, out_hbm, idx_v, rows_v, pos_v,
          gsem0, gsem1, ssem0, ssem1):
        nc = 2
        wid = lax.axis_index("s") * nc + lax.axis_index("c")
        base = wid * rows_per_w
        pltpu.sync_copy(idx_hbm.at[pl.ds(base, rows_per_w)], idx_v)
        pltpu.sync_copy(pos_hbm, pos_v)
        gsems = (gsem0, gsem1)
        ssems = (ssem0, ssem1)

        def gather(c):
            b = c % NBUF
            return pltpu.make_async_copy(
                table_hbm.at[idx_v.at[pl.ds(c * CHUNK, CHUNK)]],
                rows_v.at[b],
                gsems[b],
            )

        def store(c):
            b = c % NBUF
            return pltpu.make_async_copy(
                rows_v.at[b],
                out_hbm.at[pl.ds(base + c * CHUNK, CHUNK)],
                ssems[b],
            )

        gather(0).start()

        for c in range(nchunk):
            b = c % NBUF
            gather(c).wait()
            if c + 1 < nchunk:
                if c + 1 >= NBUF:
                    store(c + 1 - NBUF).wait()
                gather(c + 1).start()

            def seq_body(l, carry):
                p0 = pos_v[l, pl.ds(0, 16)]
                p1 = pos_v[l, pl.ds(16, 16)]
                for s in range(SEQ_PER_CHUNK):
                    r = s * SEQ + l
                    rows_v[b, r, pl.ds(0, 16)] = (
                        rows_v[b, r, pl.ds(0, 16)] + p0
                    )
                    rows_v[b, r, pl.ds(16, 16)] = (
                        rows_v[b, r, pl.ds(16, 16)] + p1
                    )
                return carry

            lax.fori_loop(0, SEQ, seq_body, 0)
            store(c).start()

        for c in range(max(0, nchunk - NBUF), nchunk):
            store(c).wait()

    return k


def kernel(x, item_emb_matrix, positional_emb):
    idx = x.reshape(ROWS).astype(jnp.int32)
    t4 = jax.lax.optimization_barrier(item_emb_matrix.reshape(250000, 128))
    item_emb_matrix = t4.reshape(1000000, 32)
    info = plsc.get_sparse_core_info()
    num_workers = info.num_cores * info.num_subcores
    rows_per_w = ROWS // num_workers
    nchunk = rows_per_w // CHUNK
    out = _build(num_workers, rows_per_w, nchunk)(
        item_emb_matrix, idx, positional_emb
    )
    return out.reshape(BATCH, SEQ, DIM)


# speedup vs baseline: 1.3844x; 1.0849x over previous
import functools

import jax
import jax.numpy as jnp
import numpy as np
from jax import lax
from jax.experimental import pallas as pl
from jax.experimental.pallas import tpu as pltpu
from jax.experimental.pallas import tpu_sc as plsc

BATCH = 4096
SEQ = 200
DIM = 32
NITEMS = 1000000
NW = 32
NCOLS = NITEMS // 128
COL_TAIL = NITEMS - NCOLS * 128
A_ITERS = 61
A_EXTRA = NCOLS - 4 * A_ITERS * NW
B_PER_W = BATCH // NW

_J = np.arange(16)

_LUT_A = np.zeros((8 + DIM, 128), np.int32)
for _k in range(8):
    _LUT_A[_k, :16] = 4 * _k + _J // 4
for _d in range(DIM):
    _LUT_A[8 + _d, :16] = (_J % 4) * 32 + _d

_LUT_B = np.zeros((3 + B_PER_W, 16), np.int32)
_LUT_B[0] = _J // 8
_LUT_B[1] = 2 + _J // 8
_LUT_B[2] = _J % 8
for _bi in range(B_PER_W):
    _LUT_B[3 + _bi] = _bi


def _detile_table():
    mesh = plsc.VectorSubcoreMesh(core_axis_name="c", subcore_axis_name="s")

    @functools.partial(
        pl.kernel,
        mesh=mesh,
        out_type=jax.ShapeDtypeStruct((NITEMS // 4, 128), jnp.float32),
        compiler_params=pltpu.CompilerParams(use_tc_tiling_on_sc=True, needs_layout_passes=False),
        scratch_types=[
            pltpu.VMEM((8 + DIM, 128), jnp.int32),
            pltpu.VMEM((DIM, 128), jnp.float32),
            pltpu.VMEM((DIM, 128), jnp.float32),
            pltpu.VMEM((DIM, 128), jnp.float32),
            pltpu.VMEM((DIM, 128), jnp.float32),
            pltpu.VMEM((DIM, 133), jnp.float32),
            pltpu.VMEM((DIM, 133), jnp.float32),
            pltpu.VMEM((DIM, 133), jnp.float32),
            pltpu.VMEM((DIM, 133), jnp.float32),
            pltpu.SemaphoreType.DMA,
            pltpu.SemaphoreType.DMA,
            pltpu.SemaphoreType.DMA,
            pltpu.SemaphoreType.DMA,
            pltpu.SemaphoreType.DMA,
            pltpu.SemaphoreType.DMA,
            pltpu.SemaphoreType.DMA,
            pltpu.SemaphoreType.DMA,
        ],
    )
    def ka(tt_hbm, lut_hbm, tail_hbm, out_hbm, lut_v,
           tb0, tb1, tb2, tb3, ob0, ob1, ob2, ob3,
           gs0, gs1, gs2, gs3, ss0, ss1, ss2, ss3):
        wid = lax.axis_index("s") * 2 + lax.axis_index("c")
        tbufs = (tb0, tb1, tb2, tb3)
        obufs = (ob0, ob1, ob2, ob3)
        gsems = (gs0, gs1, gs2, gs3)
        ssems = (ss0, ss1, ss2, ss3)

        pltpu.sync_copy(lut_hbm, lut_v)
        rowv = [lut_v[k, pl.ds(0, 16)] for k in range(8)]

        def col_of(t, b):
            return (4 * t + b) * NW + wid

        def fetch(t, b):
            col = col_of(t, b)
            return pltpu.make_async_copy(
                tt_hbm.at[:, pl.ds(col * 128, 128)], tbufs[b], gsems[b]
            )

        def flush(t, b):
            col = col_of(t, b)
            return pltpu.make_async_copy(
                obufs[b].at[:, pl.ds(0, 128)],
                out_hbm.at[pl.ds(col * 32, DIM)],
                ssems[b],
            )

        def transpose_col(tbuf, obuf, nk):
            def dbody(o, carry):
                for u in range(4):
                    d = o * 4 + u
                    cvec = lut_v[8 + d, pl.ds(0, 16)]
                    for k in range(nk):
                        v = tbuf[d, pl.ds(16 * k, 16)]
                        plsc.store_scatter(obuf, [rowv[k], cvec], v)
                return carry

            lax.fori_loop(0, DIM // 4, dbody, 0)

        for b in range(4):
            fetch(0, b).start()

        for b in range(4):
            fetch(0, b).wait()
            transpose_col(tbufs[b], obufs[b], 8)
            fetch(1, b).start()
            flush(0, b).start()

        def tbody(t, carry):
            for b in range(4):
                fetch(t, b).wait()
                flush(t - 1, b).wait()
                transpose_col(tbufs[b], obufs[b], 8)
                fetch(t + 1, b).start()
                flush(t, b).start()
            return carry

        lax.fori_loop(1, A_ITERS - 1, tbody, 0)

        for b in range(4):
            t = A_ITERS - 1
            fetch(t, b).wait()
            flush(t - 1, b).wait()
            transpose_col(tbufs[b], obufs[b], 8)
            flush(t, b).start()
        for b in range(4):
            flush(A_ITERS - 1, b).wait()

        @pl.when(wid < A_EXTRA)
        def _():
            col = 4 * A_ITERS * NW + wid
            pltpu.sync_copy(tt_hbm.at[:, pl.ds(col * 128, 128)], tb0)
            transpose_col(tb0, ob0, 8)
            pltpu.sync_copy(
                ob0.at[:, pl.ds(0, 128)], out_hbm.at[pl.ds(col * 32, DIM)]
            )

        @pl.when(wid == A_EXTRA)
        def _():
            pltpu.sync_copy(
                tail_hbm, ob1.at[pl.ds(0, COL_TAIL // 4), pl.ds(0, 128)]
            )
            pltpu.sync_copy(
                ob1.at[pl.ds(0, COL_TAIL // 4), pl.ds(0, 128)],
                out_hbm.at[pl.ds(NCOLS * 32, COL_TAIL // 4)],
            )

    return ka


def _gather_add():
    mesh = plsc.VectorSubcoreMesh(core_axis_name="c", subcore_axis_name="s")

    @functools.partial(
        pl.kernel,
        mesh=mesh,
        out_type=jax.ShapeDtypeStruct((SEQ, 4, 32, 8, 128), jnp.float32),
        compiler_params=pltpu.CompilerParams(use_tc_tiling_on_sc=False, needs_layout_passes=False),
        scratch_types=[
            pltpu.VMEM((3 + B_PER_W, 16), jnp.int32),
            pltpu.VMEM((SEQ, B_PER_W), jnp.int32),
            pltpu.VMEM((SEQ, DIM), jnp.float32),
            pltpu.VMEM((4 * B_PER_W, DIM), jnp.float32),
            pltpu.VMEM((4 * B_PER_W, DIM), jnp.float32),
            pltpu.VMEM((4, 4, 8, 133), jnp.float32),
            pltpu.VMEM((4, 4, 8, 133), jnp.float32),
            pltpu.SemaphoreType.DMA,
            pltpu.SemaphoreType.DMA,
            pltpu.SemaphoreType.DMA,
            pltpu.SemaphoreType.DMA,
        ],
    )
    def kb(tab_hbm, xt_hbm, pos_hbm, lut_hbm, out_hbm, lut_v, idx_all,
           pos_v, gb0, gb1, ob0, ob1, gs0, gs1, ss0, ss1):
        wid = lax.axis_index("s") * 2 + lax.axis_index("c")
        gbufs = (gb0, gb1)
        obufs = (ob0, ob1)
        gsems = (gs0, gs1)
        ssems = (ss0, ss1)

        pltpu.sync_copy(lut_hbm, lut_v)
        pltpu.sync_copy(xt_hbm.at[:, pl.ds(wid * B_PER_W, B_PER_W)], idx_all)
        pltpu.sync_copy(pos_hbm, pos_v)

        dt0 = lut_v[0, pl.ds(0, 16)]
        dt1 = lut_v[1, pl.ds(0, 16)]
        dsv = lut_v[2, pl.ds(0, 16)]

        NG = SEQ // 4

        def fetch(g, b):
            return [
                pltpu.make_async_copy(
                    tab_hbm.at[idx_all.at[4 * g + u]],
                    gbufs[b].at[pl.ds(u * B_PER_W, B_PER_W)],
                    gsems[b],
                )
                for u in range(4)
            ]

        def fetch_start(g, b):
            for c in fetch(g, b):
                c.start()

        def fetch_wait(g, b):
            for c in fetch(g, b):
                c.wait()

        def flush(g, b):
            return pltpu.make_async_copy(
                obufs[b].at[:, :, :, pl.ds(0, 128)],
                out_hbm.at[pl.ds(4 * g, 4), :, wid],
                ssems[b],
            )

        def process(g, gbuf, obuf):
            for u in range(4):
                l = 4 * g + u
                p0 = pos_v[l, pl.ds(0, 16)]
                p1 = pos_v[l, pl.ds(16, 16)]
                osub = obuf.at[u]

                def bbody(b4, carry, p0=p0, p1=p1, osub=osub, u=u):
                    for v in range(4):
                        bi = b4 * 4 + v
                        bvec = lut_v[3 + bi, pl.ds(0, 16)]
                        v0 = gbuf[u * B_PER_W + bi, pl.ds(0, 16)] + p0
                        v1 = gbuf[u * B_PER_W + bi, pl.ds(16, 16)] + p1
                        plsc.store_scatter(osub, [dt0, dsv, bvec], v0)
                        plsc.store_scatter(osub, [dt1, dsv, bvec], v1)
                    return carry

                lax.fori_loop(0, B_PER_W // 4, bbody, 0)

        for b in range(2):
            fetch_start(b, b)

        for b in range(2):
            fetch_wait(b, b)
            process(b, gbufs[b], obufs[b])
            fetch_start(b + 2, b)
            flush(b, b).start()

        def gbody(t, carry):
            for b in range(2):
                g = 2 * t + b
                fetch_wait(g, b)
                flush(g - 2, b).wait()
                process(g, gbufs[b], obufs[b])
                fetch_start(g + 2, b)
                flush(g, b).start()
            return carry

        lax.fori_loop(1, NG // 2 - 1, gbody, 0)

        for b in range(2):
            g = NG - 2 + b
            fetch_wait(g, b)
            flush(g - 2, b).wait()
            process(g, gbufs[b], obufs[b])
            flush(g, b).start()
        for b in range(2):
            flush(NG - 2 + b, b).wait()

    return kb


def kernel(x, item_emb_matrix, positional_emb):
    table_t = jnp.swapaxes(item_emb_matrix, 0, 1)
    lut_a = jnp.asarray(_LUT_A)
    tail_rm = item_emb_matrix[NCOLS * 128:].reshape(COL_TAIL // 4, 128)
    table_rm = _detile_table()(table_t, lut_a, tail_rm)
    table_flat = table_rm.reshape(NITEMS, DIM)
    x_t = jnp.swapaxes(x, 0, 1).astype(jnp.int32)
    lut_b = jnp.asarray(_LUT_B)
    out5 = _gather_add()(table_flat, x_t, positional_emb, lut_b)
    return out5.transpose(2, 4, 0, 1, 3).reshape(BATCH, SEQ, DIM)
